# baseline (speedup 1.0000x reference)
import functools

import jax
import jax.numpy as jnp
from jax import lax
from jax.experimental import pallas as pl
from jax.experimental.pallas import tpu as pltpu
from jax.experimental.pallas import tpu_sc as plsc

_N = 10000
_E = 640000
_H = 128
_DEPTH = 6
_NG = 500
_NX = 20000
_IN_NODE = 35
_IN_EDGE = 40

_NC = 2
_NS = 16
_NW = _NC * _NS
_EPW = _E // _NW
_CH = 80
_NCHUNK = _EPW // _CH

_TE = 1280
_TN = 512
_NPAD = 10240
_GPAD = 512

_NBUF = 5
_NOUTER = _NCHUNK // _NBUF

_sc_mesh = plsc.VectorSubcoreMesh(core_axis_name="c", subcore_axis_name="s")


@functools.partial(
    pl.kernel,
    mesh=_sc_mesh,
    out_type=jax.ShapeDtypeStruct((_E, _H), jnp.float32),
    scratch_types=[
        pltpu.VMEM((_NCHUNK, _CH), jnp.int32),
        pltpu.VMEM((_NBUF, _CH, _H), jnp.float32),
        [pltpu.SemaphoreType.DMA] * _NBUF,
        [pltpu.SemaphoreType.DMA] * _NBUF,
    ],
)
def _gather_rows(table_hbm, idx_hbm, out_hbm, idx_vm, rows_v, sem_g, sem_s):
    wid = lax.axis_index("s") * _NC + lax.axis_index("c")
    base = wid * _EPW
    pltpu.sync_copy(idx_hbm.at[wid], idx_vm)

    def outer(g, carry):
        for b in range(_NBUF):
            ci = g * _NBUF + b

            @pl.when(g > 0)
            def _():
                pltpu.make_async_copy(
                    rows_v.at[b], out_hbm.at[pl.ds(base, _CH)], sem_s[b]).wait()

            pltpu.async_copy(table_hbm.at[idx_vm.at[ci]], rows_v.at[b], sem_g[b])
        for b in range(_NBUF):
            ci = g * _NBUF + b
            off = base + ci * _CH
            pltpu.make_async_copy(
                table_hbm.at[idx_vm.at[ci]], rows_v.at[b], sem_g[b]).wait()
            pltpu.async_copy(rows_v.at[b], out_hbm.at[pl.ds(off, _CH)], sem_s[b])
        return carry

    lax.fori_loop(0, _NOUTER, outer, 0)
    for b in range(_NBUF):
        pltpu.make_async_copy(
            rows_v.at[b], out_hbm.at[pl.ds(base, _CH)], sem_s[b]).wait()


_CH_S = 40
_NCHUNK_S = _EPW // _CH_S
_NBUF_S = 5


@functools.partial(
    pl.kernel,
    mesh=_sc_mesh,
    out_type=jax.ShapeDtypeStruct((_NC, _N, _H), jnp.float32),
    scratch_types=[
        pltpu.VMEM((_NBUF_S, _CH_S), jnp.int32),
        pltpu.VMEM((_NBUF_S, _CH_S, _H), jnp.float32),
        pltpu.VMEM_SHARED((_N, _H), jnp.float32),
        [pltpu.SemaphoreType.DMA] * _NBUF_S,
        [pltpu.SemaphoreType.DMA] * _NBUF_S,
    ],
)
def _scatter_add_rows(rows_hbm, idx_hbm, zeros_hbm, out_hbm, idx_vm, rows_v,
                      acc_sh, sem_l, sem_a):
    cid = lax.axis_index("c")
    sid = lax.axis_index("s")
    wid = sid * _NC + cid
    base = wid * _EPW

    @pl.when(sid == 0)
    def _():
        pltpu.sync_copy(zeros_hbm, acc_sh)

    plsc.subcore_barrier()

    def outer(g, carry):
        for b in range(_NBUF_S):
            ci = g * _NBUF_S + b

            @pl.when(g > 0)
            def _():
                pltpu.make_async_copy(
                    rows_v.at[b], acc_sh.at[idx_vm.at[b]], sem_a[b]).wait()

            off = base + ci * _CH_S
            pltpu.async_copy(idx_hbm.at[wid, ci], idx_vm.at[b], sem_l[b])
            pltpu.async_copy(rows_hbm.at[pl.ds(off, _CH_S)], rows_v.at[b], sem_l[b])
        for b in range(_NBUF_S):
            ci = g * _NBUF_S + b
            off = base + ci * _CH_S
            pltpu.make_async_copy(
                idx_hbm.at[wid, ci], idx_vm.at[b], sem_l[b]).wait()
            pltpu.make_async_copy(
                rows_hbm.at[pl.ds(off, _CH_S)], rows_v.at[b], sem_l[b]).wait()
            pltpu.async_copy(rows_v.at[b], acc_sh.at[idx_vm.at[b]], sem_a[b],
                             add=True)
        return carry

    lax.fori_loop(0, _NCHUNK_S // _NBUF_S, outer, 0)
    for b in range(_NBUF_S):
        pltpu.make_async_copy(
            rows_v.at[b], acc_sh.at[idx_vm.at[b]], sem_a[b]).wait()

    plsc.subcore_barrier()

    @pl.when(sid == 0)
    def _():
        pltpu.sync_copy(acc_sh, out_hbm.at[cid])


def _prologue_body(ex_ref, ns_ref, wi_ref, wh_ref, bth_ref, p1_ref):
    binput = jnp.dot(ex_ref[...], wi_ref[...], preferred_element_type=jnp.float32)
    bth_ref[...] = binput + jnp.dot(ns_ref[...], wh_ref[...], preferred_element_type=jnp.float32)
    p1_ref[...] = jnp.dot(jnp.maximum(binput, 0.0), wh_ref[...],
                          preferred_element_type=jnp.float32)


def _prologue(edge_x, nt_src, W_i, W_h):
    grid = _E // _TE
    return pl.pallas_call(
        _prologue_body,
        grid=(grid,),
        in_specs=[
            pl.BlockSpec((_TE, _IN_EDGE), lambda i: (i, 0)),
            pl.BlockSpec((_TE, _H), lambda i: (i, 0)),
            pl.BlockSpec((_IN_EDGE, _H), lambda i: (0, 0)),
            pl.BlockSpec((_H, _H), lambda i: (0, 0)),
        ],
        out_specs=[
            pl.BlockSpec((_TE, _H), lambda i: (i, 0)),
            pl.BlockSpec((_TE, _H), lambda i: (i, 0)),
        ],
        out_shape=[
            jax.ShapeDtypeStruct((_E, _H), jnp.float32),
            jax.ShapeDtypeStruct((_E, _H), jnp.float32),
        ],
    )(edge_x, nt_src, W_i, W_h)


def _iter_body(p2_ref, bth2_ref, agg2_ref, wh_ref, out2_ref):
    p2 = p2_ref[...]
    pswap = jnp.concatenate([p2[:, _H:], p2[:, :_H]], axis=1)
    g2 = jnp.maximum(bth2_ref[...] + agg2_ref[...] - pswap, 0.0)
    a = jnp.dot(g2[:, :_H], wh_ref[...], preferred_element_type=jnp.float32)
    b = jnp.dot(g2[:, _H:], wh_ref[...], preferred_element_type=jnp.float32)
    out2_ref[...] = jnp.concatenate([a, b], axis=1)


def _iter_step(p2, bth2, agg2, W_h):
    grid = (_E // 2) // (_TE // 2)
    t2 = _TE // 2
    return pl.pallas_call(
        _iter_body,
        grid=(grid,),
        in_specs=[
            pl.BlockSpec((t2, 2 * _H), lambda i: (i, 0)),
            pl.BlockSpec((t2, 2 * _H), lambda i: (i, 0)),
            pl.BlockSpec((t2, 2 * _H), lambda i: (i, 0)),
            pl.BlockSpec((_H, _H), lambda i: (0, 0)),
        ],
        out_specs=pl.BlockSpec((t2, 2 * _H), lambda i: (i, 0)),
        out_shape=jax.ShapeDtypeStruct((_E // 2, 2 * _H), jnp.float32),
    )(p2, bth2, agg2, W_h)


def _final_body(p2_ref, bth2_ref, agg2_ref, out2_ref):
    p2 = p2_ref[...]
    pswap = jnp.concatenate([p2[:, _H:], p2[:, :_H]], axis=1)
    out2_ref[...] = jnp.maximum(bth2_ref[...] + agg2_ref[...] - pswap, 0.0)


def _final_step(p2, bth2, agg2):
    grid = (_E // 2) // (_TE // 2)
    t2 = _TE // 2
    return pl.pallas_call(
        _final_body,
        grid=(grid,),
        in_specs=[
            pl.BlockSpec((t2, 2 * _H), lambda i: (i, 0)),
            pl.BlockSpec((t2, 2 * _H), lambda i: (i, 0)),
            pl.BlockSpec((t2, 2 * _H), lambda i: (i, 0)),
        ],
        out_specs=pl.BlockSpec((t2, 2 * _H), lambda i: (i, 0)),
        out_shape=jax.ShapeDtypeStruct((_E // 2, 2 * _H), jnp.float32),
    )(p2, bth2, agg2)


def _readout_body(x_ref, nei_ref, gid_ref, woa_ref, wob_ref, bo_ref, s_ref, c_ref):
    i = pl.program_id(0)

    @pl.when(i == 0)
    def _():
        s_ref[...] = jnp.zeros_like(s_ref)
        c_ref[...] = jnp.zeros_like(c_ref)

    h = jnp.dot(x_ref[...], woa_ref[...], preferred_element_type=jnp.float32)
    h = h + jnp.dot(nei_ref[...], wob_ref[...], preferred_element_type=jnp.float32)
    h = jnp.maximum(h + bo_ref[...], 0.0)
    gidb = jnp.broadcast_to(gid_ref[...], (_GPAD, _TN))
    iota_g = lax.broadcasted_iota(jnp.int32, (_GPAD, _TN), 0)
    lane = i * _TN + lax.broadcasted_iota(jnp.int32, (_GPAD, _TN), 1)
    oh = jnp.where((iota_g == gidb) & (lane < _N), 1.0, 0.0)
    s_ref[...] += jnp.dot(oh, h, preferred_element_type=jnp.float32)
    c_ref[...] += jnp.dot(oh, jnp.ones((_TN, _H), jnp.float32),
                          preferred_element_type=jnp.float32)


def _readout(xp, neip, gidp, woa, wob, bo):
    grid = _NPAD // _TN
    return pl.pallas_call(
        _readout_body,
        grid=(grid,),
        in_specs=[
            pl.BlockSpec((_TN, _IN_NODE), lambda i: (i, 0)),
            pl.BlockSpec((_TN, _H), lambda i: (i, 0)),
            pl.BlockSpec((None, 1, _TN), lambda i: (i, 0, 0)),
            pl.BlockSpec((_IN_NODE, _H), lambda i: (0, 0)),
            pl.BlockSpec((_H, _H), lambda i: (0, 0)),
            pl.BlockSpec((1, _H), lambda i: (0, 0)),
        ],
        out_specs=[
            pl.BlockSpec((_GPAD, _H), lambda i: (0, 0)),
            pl.BlockSpec((_GPAD, _H), lambda i: (0, 0)),
        ],
        out_shape=[
            jax.ShapeDtypeStruct((_GPAD, _H), jnp.float32),
            jax.ShapeDtypeStruct((_GPAD, _H), jnp.float32),
        ],
    )(xp, neip, gidp, woa, wob, bo)


def kernel(x, edge_x, edge_index, tree_mess, tree_mess_source_edges,
           tree_mess_target_edges, graph_ids, W_i, W_h, W_o, b_o):
    src = edge_index[0]
    dst = edge_index[1]
    tmt = tree_mess_target_edges

    iotax = jnp.arange(_NX, dtype=jnp.int32)
    win = jnp.full((_E,), -1, jnp.int32).at[tmt].max(iotax)
    keep_b = win[tmt] == iotax
    rows = tree_mess[tree_mess_source_edges] * keep_b.astype(jnp.float32)[:, None]
    node_t = jnp.zeros((_N, _H), jnp.float32).at[dst[tmt]].add(rows)
    sidx = jnp.full((_E,), -1, jnp.int32).at[tmt ^ 1].max(
        jnp.where(keep_b, iotax, -1))
    idx_mod = jnp.where(sidx >= 0, _N + sidx, src)
    ext_tbl = jnp.concatenate([node_t, node_t[src[tmt ^ 1]] - rows], axis=0)
    zeros_nh = jnp.zeros((_N, _H), jnp.float32)
    src3 = src.reshape(_NW, _NCHUNK, _CH)
    dst3 = dst.reshape(_NW, _NCHUNK_S, _CH_S)

    aggt = _gather_rows(ext_tbl, idx_mod.reshape(_NW, _NCHUNK, _CH))
    bth, p = _prologue(edge_x, aggt, W_i, W_h)
    bth2 = bth.reshape(_E // 2, 2 * _H)

    for _ in range(_DEPTH - 2):
        parts = _scatter_add_rows(p, dst3, zeros_nh)
        tbl = parts[0] + parts[1]
        agg = _gather_rows(tbl, src3)
        p = _iter_step(p.reshape(_E // 2, 2 * _H), bth2,
                       agg.reshape(_E // 2, 2 * _H), W_h).reshape(_E, _H)

    parts = _scatter_add_rows(p, dst3, zeros_nh)
    tbl = parts[0] + parts[1]
    agg = _gather_rows(tbl, src3)
    g6 = _final_step(p.reshape(_E // 2, 2 * _H), bth2,
                     agg.reshape(_E // 2, 2 * _H)).reshape(_E, _H)

    parts = _scatter_add_rows(g6, dst3, zeros_nh)
    nei = parts[0] + parts[1] + node_t

    xp = jnp.pad(x, ((0, _NPAD - _N), (0, 0)))
    neip = jnp.pad(nei, ((0, _NPAD - _N), (0, 0)))
    gidp = jnp.pad(graph_ids, (0, _NPAD - _N),
                   constant_values=_NG).reshape(_NPAD // _TN, 1, _TN)
    s, c = _readout(xp, neip, gidp, W_o[:_IN_NODE], W_o[_IN_NODE:],
                    b_o.reshape(1, _H))
    return s[:_NG] / jnp.maximum(c[:_NG, :1], 1.0)


# speedup vs baseline: 1.0885x; 1.0885x over previous
import functools

import jax
import jax.numpy as jnp
from jax import lax
from jax.experimental import pallas as pl
from jax.experimental.pallas import tpu as pltpu
from jax.experimental.pallas import tpu_sc as plsc

_N = 10000
_E = 640000
_H = 128
_DEPTH = 6
_NG = 500
_NX = 20000
_IN_NODE = 35
_IN_EDGE = 40

_NC = 2
_NS = 16
_NW = _NC * _NS
_EPW = _E // _NW
_CH = 80
_NCHUNK = _EPW // _CH

_TE = 2560
_TN = 512
_NPAD = 10240
_GPAD = 512

_NBUF = 5
_NOUTER = _NCHUNK // _NBUF

_sc_mesh = plsc.VectorSubcoreMesh(core_axis_name="c", subcore_axis_name="s")


@functools.partial(
    pl.kernel,
    mesh=_sc_mesh,
    out_type=jax.ShapeDtypeStruct((_E, _H), jnp.float32),
    scratch_types=[
        pltpu.VMEM((_NCHUNK, _CH), jnp.int32),
        pltpu.VMEM((_NBUF, _CH, _H), jnp.float32),
        [pltpu.SemaphoreType.DMA] * _NBUF,
        [pltpu.SemaphoreType.DMA] * _NBUF,
    ],
)
def _gather_rows(table_hbm, idx_hbm, out_hbm, idx_vm, rows_v, sem_g, sem_s):
    wid = lax.axis_index("s") * _NC + lax.axis_index("c")
    base = wid * _EPW
    pltpu.sync_copy(idx_hbm.at[wid], idx_vm)

    def outer(g, carry):
        for b in range(_NBUF):
            ci = g * _NBUF + b

            @pl.when(g > 0)
            def _():
                pltpu.make_async_copy(
                    rows_v.at[b], out_hbm.at[pl.ds(base, _CH)], sem_s[b]).wait()

            pltpu.async_copy(table_hbm.at[idx_vm.at[ci]], rows_v.at[b], sem_g[b])
        for b in range(_NBUF):
            ci = g * _NBUF + b
            off = base + ci * _CH
            pltpu.make_async_copy(
                table_hbm.at[idx_vm.at[ci]], rows_v.at[b], sem_g[b]).wait()
            pltpu.async_copy(rows_v.at[b], out_hbm.at[pl.ds(off, _CH)], sem_s[b])
        return carry

    lax.fori_loop(0, _NOUTER, outer, 0)
    for b in range(_NBUF):
        pltpu.make_async_copy(
            rows_v.at[b], out_hbm.at[pl.ds(base, _CH)], sem_s[b]).wait()


_CH_S = 40
_NCHUNK_S = _EPW // _CH_S
_NBUF_S = 5


@functools.partial(
    pl.kernel,
    mesh=_sc_mesh,
    out_type=jax.ShapeDtypeStruct((_NC, _N, _H), jnp.float32),
    scratch_types=[
        pltpu.VMEM((_NBUF_S, _CH_S), jnp.int32),
        pltpu.VMEM((_NBUF_S, _CH_S, _H), jnp.float32),
        pltpu.VMEM_SHARED((_N, _H), jnp.float32),
        [pltpu.SemaphoreType.DMA] * _NBUF_S,
        [pltpu.SemaphoreType.DMA] * _NBUF_S,
    ],
)
def _scatter_add_rows(rows_hbm, idx_hbm, zeros_hbm, out_hbm, idx_vm, rows_v,
                      acc_sh, sem_l, sem_a):
    cid = lax.axis_index("c")
    sid = lax.axis_index("s")
    wid = sid * _NC + cid
    base = wid * _EPW

    @pl.when(sid == 0)
    def _():
        pltpu.sync_copy(zeros_hbm, acc_sh)

    plsc.subcore_barrier()

    def outer(g, carry):
        for b in range(_NBUF_S):
            ci = g * _NBUF_S + b

            @pl.when(g > 0)
            def _():
                pltpu.make_async_copy(
                    rows_v.at[b], acc_sh.at[idx_vm.at[b]], sem_a[b]).wait()

            off = base + ci * _CH_S
            pltpu.async_copy(idx_hbm.at[wid, ci], idx_vm.at[b], sem_l[b])
            pltpu.async_copy(rows_hbm.at[pl.ds(off, _CH_S)], rows_v.at[b], sem_l[b])
        for b in range(_NBUF_S):
            ci = g * _NBUF_S + b
            off = base + ci * _CH_S
            pltpu.make_async_copy(
                idx_hbm.at[wid, ci], idx_vm.at[b], sem_l[b]).wait()
            pltpu.make_async_copy(
                rows_hbm.at[pl.ds(off, _CH_S)], rows_v.at[b], sem_l[b]).wait()
            pltpu.async_copy(rows_v.at[b], acc_sh.at[idx_vm.at[b]], sem_a[b],
                             add=True)
        return carry

    lax.fori_loop(0, _NCHUNK_S // _NBUF_S, outer, 0)
    for b in range(_NBUF_S):
        pltpu.make_async_copy(
            rows_v.at[b], acc_sh.at[idx_vm.at[b]], sem_a[b]).wait()

    plsc.subcore_barrier()

    @pl.when(sid == 0)
    def _():
        pltpu.sync_copy(acc_sh, out_hbm.at[cid])


def _prologue_body(ex_ref, ns_ref, wi_ref, wh_ref, bth_ref, p1_ref):
    binput = jnp.dot(ex_ref[...], wi_ref[...], preferred_element_type=jnp.float32)
    bth_ref[...] = binput + jnp.dot(ns_ref[...], wh_ref[...], preferred_element_type=jnp.float32)
    p1_ref[...] = jnp.dot(jnp.maximum(binput, 0.0), wh_ref[...],
                          preferred_element_type=jnp.float32)


def _prologue(edge_x, nt_src, W_i, W_h):
    grid = _E // _TE
    return pl.pallas_call(
        _prologue_body,
        grid=(grid,),
        in_specs=[
            pl.BlockSpec((_TE, _IN_EDGE), lambda i: (i, 0)),
            pl.BlockSpec((_TE, _H), lambda i: (i, 0)),
            pl.BlockSpec((_IN_EDGE, _H), lambda i: (0, 0)),
            pl.BlockSpec((_H, _H), lambda i: (0, 0)),
        ],
        out_specs=[
            pl.BlockSpec((_TE, _H), lambda i: (i, 0)),
            pl.BlockSpec((_TE, _H), lambda i: (i, 0)),
        ],
        out_shape=[
            jax.ShapeDtypeStruct((_E, _H), jnp.float32),
            jax.ShapeDtypeStruct((_E, _H), jnp.float32),
        ],
    )(edge_x, nt_src, W_i, W_h)


def _iter_body(p2_ref, bth2_ref, agg2_ref, wh_ref, out2_ref):
    p2 = p2_ref[...]
    pswap = jnp.concatenate([p2[:, _H:], p2[:, :_H]], axis=1)
    g2 = jnp.maximum(bth2_ref[...] + agg2_ref[...] - pswap, 0.0)
    a = jnp.dot(g2[:, :_H], wh_ref[...], preferred_element_type=jnp.float32)
    b = jnp.dot(g2[:, _H:], wh_ref[...], preferred_element_type=jnp.float32)
    out2_ref[...] = jnp.concatenate([a, b], axis=1)


def _iter_step(p2, bth2, agg2, W_h):
    grid = (_E // 2) // (_TE // 2)
    t2 = _TE // 2
    return pl.pallas_call(
        _iter_body,
        grid=(grid,),
        in_specs=[
            pl.BlockSpec((t2, 2 * _H), lambda i: (i, 0)),
            pl.BlockSpec((t2, 2 * _H), lambda i: (i, 0)),
            pl.BlockSpec((t2, 2 * _H), lambda i: (i, 0)),
            pl.BlockSpec((_H, _H), lambda i: (0, 0)),
        ],
        out_specs=pl.BlockSpec((t2, 2 * _H), lambda i: (i, 0)),
        out_shape=jax.ShapeDtypeStruct((_E // 2, 2 * _H), jnp.float32),
    )(p2, bth2, agg2, W_h)


def _final_body(p2_ref, bth2_ref, agg2_ref, out2_ref):
    p2 = p2_ref[...]
    pswap = jnp.concatenate([p2[:, _H:], p2[:, :_H]], axis=1)
    out2_ref[...] = jnp.maximum(bth2_ref[...] + agg2_ref[...] - pswap, 0.0)


def _final_step(p2, bth2, agg2):
    grid = (_E // 2) // (_TE // 2)
    t2 = _TE // 2
    return pl.pallas_call(
        _final_body,
        grid=(grid,),
        in_specs=[
            pl.BlockSpec((t2, 2 * _H), lambda i: (i, 0)),
            pl.BlockSpec((t2, 2 * _H), lambda i: (i, 0)),
            pl.BlockSpec((t2, 2 * _H), lambda i: (i, 0)),
        ],
        out_specs=pl.BlockSpec((t2, 2 * _H), lambda i: (i, 0)),
        out_shape=jax.ShapeDtypeStruct((_E // 2, 2 * _H), jnp.float32),
    )(p2, bth2, agg2)


def _readout_body(x_ref, nei_ref, gid_ref, woa_ref, wob_ref, bo_ref, s_ref, c_ref):
    i = pl.program_id(0)

    @pl.when(i == 0)
    def _():
        s_ref[...] = jnp.zeros_like(s_ref)
        c_ref[...] = jnp.zeros_like(c_ref)

    h = jnp.dot(x_ref[...], woa_ref[...], preferred_element_type=jnp.float32)
    h = h + jnp.dot(nei_ref[...], wob_ref[...], preferred_element_type=jnp.float32)
    h = jnp.maximum(h + bo_ref[...], 0.0)
    gidb = jnp.broadcast_to(gid_ref[...], (_GPAD, _TN))
    iota_g = lax.broadcasted_iota(jnp.int32, (_GPAD, _TN), 0)
    lane = i * _TN + lax.broadcasted_iota(jnp.int32, (_GPAD, _TN), 1)
    oh = jnp.where((iota_g == gidb) & (lane < _N), 1.0, 0.0)
    s_ref[...] += jnp.dot(oh, h, preferred_element_type=jnp.float32)
    c_ref[...] += jnp.dot(oh, jnp.ones((_TN, _H), jnp.float32),
                          preferred_element_type=jnp.float32)


def _readout(xp, neip, gidp, woa, wob, bo):
    grid = _NPAD // _TN
    return pl.pallas_call(
        _readout_body,
        grid=(grid,),
        in_specs=[
            pl.BlockSpec((_TN, _IN_NODE), lambda i: (i, 0)),
            pl.BlockSpec((_TN, _H), lambda i: (i, 0)),
            pl.BlockSpec((None, 1, _TN), lambda i: (i, 0, 0)),
            pl.BlockSpec((_IN_NODE, _H), lambda i: (0, 0)),
            pl.BlockSpec((_H, _H), lambda i: (0, 0)),
            pl.BlockSpec((1, _H), lambda i: (0, 0)),
        ],
        out_specs=[
            pl.BlockSpec((_GPAD, _H), lambda i: (0, 0)),
            pl.BlockSpec((_GPAD, _H), lambda i: (0, 0)),
        ],
        out_shape=[
            jax.ShapeDtypeStruct((_GPAD, _H), jnp.float32),
            jax.ShapeDtypeStruct((_GPAD, _H), jnp.float32),
        ],
    )(xp, neip, gidp, woa, wob, bo)


def kernel(x, edge_x, edge_index, tree_mess, tree_mess_source_edges,
           tree_mess_target_edges, graph_ids, W_i, W_h, W_o, b_o):
    src = edge_index[0]
    dst = edge_index[1]
    tmt = tree_mess_target_edges

    iotax = jnp.arange(_NX, dtype=jnp.int32)
    win = jnp.full((_E,), -1, jnp.int32).at[tmt].max(iotax)
    keep_b = win[tmt] == iotax
    rows = tree_mess[tree_mess_source_edges] * keep_b.astype(jnp.float32)[:, None]
    node_t = jnp.zeros((_N, _H), jnp.float32).at[dst[tmt]].add(rows)
    sidx = jnp.full((_E,), -1, jnp.int32).at[tmt ^ 1].max(
        jnp.where(keep_b, iotax, -1))
    idx_mod = jnp.where(sidx >= 0, _N + sidx, src)
    ext_tbl = jnp.concatenate([node_t, node_t[src[tmt ^ 1]] - rows], axis=0)
    zeros_nh = jnp.zeros((_N, _H), jnp.float32)
    src3 = src.reshape(_NW, _NCHUNK, _CH)
    dst3 = dst.reshape(_NW, _NCHUNK_S, _CH_S)

    aggt = _gather_rows(ext_tbl, idx_mod.reshape(_NW, _NCHUNK, _CH))
    bth, p = _prologue(edge_x, aggt, W_i, W_h)
    bth2 = bth.reshape(_E // 2, 2 * _H)

    for _ in range(_DEPTH - 2):
        parts = _scatter_add_rows(p, dst3, zeros_nh)
        tbl = parts[0] + parts[1]
        agg = _gather_rows(tbl, src3)
        p = _iter_step(p.reshape(_E // 2, 2 * _H), bth2,
                       agg.reshape(_E // 2, 2 * _H), W_h).reshape(_E, _H)

    parts = _scatter_add_rows(p, dst3, zeros_nh)
    tbl = parts[0] + parts[1]
    agg = _gather_rows(tbl, src3)
    g6 = _final_step(p.reshape(_E // 2, 2 * _H), bth2,
                     agg.reshape(_E // 2, 2 * _H)).reshape(_E, _H)

    parts = _scatter_add_rows(g6, dst3, zeros_nh)
    nei = parts[0] + parts[1] + node_t

    xp = jnp.pad(x, ((0, _NPAD - _N), (0, 0)))
    neip = jnp.pad(nei, ((0, _NPAD - _N), (0, 0)))
    gidp = jnp.pad(graph_ids, (0, _NPAD - _N),
                   constant_values=_NG).reshape(_NPAD // _TN, 1, _TN)
    s, c = _readout(xp, neip, gidp, W_o[:_IN_NODE], W_o[_IN_NODE:],
                    b_o.reshape(1, _H))
    return s[:_NG] / jnp.maximum(c[:_NG, :1], 1.0)


# speedup vs baseline: 1.1062x; 1.0162x over previous
import functools

import jax
import jax.numpy as jnp
from jax import lax
from jax.experimental import pallas as pl
from jax.experimental.pallas import tpu as pltpu
from jax.experimental.pallas import tpu_sc as plsc

_N = 10000
_E = 640000
_H = 128
_DEPTH = 6
_NG = 500
_NX = 20000
_IN_NODE = 35
_IN_EDGE = 40

_NC = 2
_NS = 16
_NW = _NC * _NS
_EPW = _E // _NW
_CH = 80
_NCHUNK = _EPW // _CH

_TE = 5120
_TN = 512
_NPAD = 10240
_GPAD = 512

_NBUF = 5
_NOUTER = _NCHUNK // _NBUF

_sc_mesh = plsc.VectorSubcoreMesh(core_axis_name="c", subcore_axis_name="s")


@functools.partial(
    pl.kernel,
    mesh=_sc_mesh,
    out_type=jax.ShapeDtypeStruct((_E, _H), jnp.float32),
    scratch_types=[
        pltpu.VMEM((_NCHUNK, _CH), jnp.int32),
        pltpu.VMEM((_NBUF, _CH, _H), jnp.float32),
        [pltpu.SemaphoreType.DMA] * _NBUF,
        [pltpu.SemaphoreType.DMA] * _NBUF,
    ],
)
def _gather_rows(table_hbm, idx_hbm, out_hbm, idx_vm, rows_v, sem_g, sem_s):
    wid = lax.axis_index("s") * _NC + lax.axis_index("c")
    base = wid * _EPW
    pltpu.sync_copy(idx_hbm.at[wid], idx_vm)

    def outer(g, carry):
        for b in range(_NBUF):
            ci = g * _NBUF + b

            @pl.when(g > 0)
            def _():
                pltpu.make_async_copy(
                    rows_v.at[b], out_hbm.at[pl.ds(base, _CH)], sem_s[b]).wait()

            pltpu.async_copy(table_hbm.at[idx_vm.at[ci]], rows_v.at[b], sem_g[b])
        for b in range(_NBUF):
            ci = g * _NBUF + b
            off = base + ci * _CH
            pltpu.make_async_copy(
                table_hbm.at[idx_vm.at[ci]], rows_v.at[b], sem_g[b]).wait()
            pltpu.async_copy(rows_v.at[b], out_hbm.at[pl.ds(off, _CH)], sem_s[b])
        return carry

    lax.fori_loop(0, _NOUTER, outer, 0)
    for b in range(_NBUF):
        pltpu.make_async_copy(
            rows_v.at[b], out_hbm.at[pl.ds(base, _CH)], sem_s[b]).wait()


_CH_S = 40
_NCHUNK_S = _EPW // _CH_S
_NBUF_S = 5


@functools.partial(
    pl.kernel,
    mesh=_sc_mesh,
    out_type=jax.ShapeDtypeStruct((_NC, _N, _H), jnp.float32),
    scratch_types=[
        pltpu.VMEM((_NBUF_S, _CH_S), jnp.int32),
        pltpu.VMEM((_NBUF_S, _CH_S, _H), jnp.float32),
        pltpu.VMEM_SHARED((_N, _H), jnp.float32),
        [pltpu.SemaphoreType.DMA] * _NBUF_S,
        [pltpu.SemaphoreType.DMA] * _NBUF_S,
    ],
)
def _scatter_add_rows(rows_hbm, idx_hbm, zeros_hbm, out_hbm, idx_vm, rows_v,
                      acc_sh, sem_l, sem_a):
    cid = lax.axis_index("c")
    sid = lax.axis_index("s")
    wid = sid * _NC + cid
    base = wid * _EPW

    @pl.when(sid == 0)
    def _():
        pltpu.sync_copy(zeros_hbm, acc_sh)

    plsc.subcore_barrier()

    def outer(g, carry):
        for b in range(_NBUF_S):
            ci = g * _NBUF_S + b

            @pl.when(g > 0)
            def _():
                pltpu.make_async_copy(
                    rows_v.at[b], acc_sh.at[idx_vm.at[b]], sem_a[b]).wait()

            off = base + ci * _CH_S
            pltpu.async_copy(idx_hbm.at[wid, ci], idx_vm.at[b], sem_l[b])
            pltpu.async_copy(rows_hbm.at[pl.ds(off, _CH_S)], rows_v.at[b], sem_l[b])
        for b in range(_NBUF_S):
            ci = g * _NBUF_S + b
            off = base + ci * _CH_S
            pltpu.make_async_copy(
                idx_hbm.at[wid, ci], idx_vm.at[b], sem_l[b]).wait()
            pltpu.make_async_copy(
                rows_hbm.at[pl.ds(off, _CH_S)], rows_v.at[b], sem_l[b]).wait()
            pltpu.async_copy(rows_v.at[b], acc_sh.at[idx_vm.at[b]], sem_a[b],
                             add=True)
        return carry

    lax.fori_loop(0, _NCHUNK_S // _NBUF_S, outer, 0)
    for b in range(_NBUF_S):
        pltpu.make_async_copy(
            rows_v.at[b], acc_sh.at[idx_vm.at[b]], sem_a[b]).wait()

    plsc.subcore_barrier()

    @pl.when(sid == 0)
    def _():
        pltpu.sync_copy(acc_sh, out_hbm.at[cid])


def _prologue_body(ex_ref, ns_ref, wi_ref, wh_ref, bth_ref, p1_ref):
    binput = jnp.dot(ex_ref[...], wi_ref[...], preferred_element_type=jnp.float32)
    bth_ref[...] = binput + jnp.dot(ns_ref[...], wh_ref[...], preferred_element_type=jnp.float32)
    p1_ref[...] = jnp.dot(jnp.maximum(binput, 0.0), wh_ref[...],
                          preferred_element_type=jnp.float32)


def _prologue(edge_x, nt_src, W_i, W_h):
    grid = _E // _TE
    return pl.pallas_call(
        _prologue_body,
        grid=(grid,),
        in_specs=[
            pl.BlockSpec((_TE, _IN_EDGE), lambda i: (i, 0)),
            pl.BlockSpec((_TE, _H), lambda i: (i, 0)),
            pl.BlockSpec((_IN_EDGE, _H), lambda i: (0, 0)),
            pl.BlockSpec((_H, _H), lambda i: (0, 0)),
        ],
        out_specs=[
            pl.BlockSpec((_TE, _H), lambda i: (i, 0)),
            pl.BlockSpec((_TE, _H), lambda i: (i, 0)),
        ],
        out_shape=[
            jax.ShapeDtypeStruct((_E, _H), jnp.float32),
            jax.ShapeDtypeStruct((_E, _H), jnp.float32),
        ],
    )(edge_x, nt_src, W_i, W_h)


def _iter_body(p2_ref, bth2_ref, agg2_ref, wh_ref, out2_ref):
    p2 = p2_ref[...]
    pswap = jnp.concatenate([p2[:, _H:], p2[:, :_H]], axis=1)
    g2 = jnp.maximum(bth2_ref[...] + agg2_ref[...] - pswap, 0.0)
    a = jnp.dot(g2[:, :_H], wh_ref[...], preferred_element_type=jnp.float32)
    b = jnp.dot(g2[:, _H:], wh_ref[...], preferred_element_type=jnp.float32)
    out2_ref[...] = jnp.concatenate([a, b], axis=1)


def _iter_step(p2, bth2, agg2, W_h):
    grid = (_E // 2) // (_TE // 2)
    t2 = _TE // 2
    return pl.pallas_call(
        _iter_body,
        grid=(grid,),
        in_specs=[
            pl.BlockSpec((t2, 2 * _H), lambda i: (i, 0)),
            pl.BlockSpec((t2, 2 * _H), lambda i: (i, 0)),
            pl.BlockSpec((t2, 2 * _H), lambda i: (i, 0)),
            pl.BlockSpec((_H, _H), lambda i: (0, 0)),
        ],
        out_specs=pl.BlockSpec((t2, 2 * _H), lambda i: (i, 0)),
        out_shape=jax.ShapeDtypeStruct((_E // 2, 2 * _H), jnp.float32),
    )(p2, bth2, agg2, W_h)


def _final_body(p2_ref, bth2_ref, agg2_ref, out2_ref):
    p2 = p2_ref[...]
    pswap = jnp.concatenate([p2[:, _H:], p2[:, :_H]], axis=1)
    out2_ref[...] = jnp.maximum(bth2_ref[...] + agg2_ref[...] - pswap, 0.0)


def _final_step(p2, bth2, agg2):
    grid = (_E // 2) // (_TE // 2)
    t2 = _TE // 2
    return pl.pallas_call(
        _final_body,
        grid=(grid,),
        in_specs=[
            pl.BlockSpec((t2, 2 * _H), lambda i: (i, 0)),
            pl.BlockSpec((t2, 2 * _H), lambda i: (i, 0)),
            pl.BlockSpec((t2, 2 * _H), lambda i: (i, 0)),
        ],
        out_specs=pl.BlockSpec((t2, 2 * _H), lambda i: (i, 0)),
        out_shape=jax.ShapeDtypeStruct((_E // 2, 2 * _H), jnp.float32),
    )(p2, bth2, agg2)


def _readout_body(x_ref, nei_ref, gid_ref, woa_ref, wob_ref, bo_ref, s_ref, c_ref):
    i = pl.program_id(0)

    @pl.when(i == 0)
    def _():
        s_ref[...] = jnp.zeros_like(s_ref)
        c_ref[...] = jnp.zeros_like(c_ref)

    h = jnp.dot(x_ref[...], woa_ref[...], preferred_element_type=jnp.float32)
    h = h + jnp.dot(nei_ref[...], wob_ref[...], preferred_element_type=jnp.float32)
    h = jnp.maximum(h + bo_ref[...], 0.0)
    gidb = jnp.broadcast_to(gid_ref[...], (_GPAD, _TN))
    iota_g = lax.broadcasted_iota(jnp.int32, (_GPAD, _TN), 0)
    lane = i * _TN + lax.broadcasted_iota(jnp.int32, (_GPAD, _TN), 1)
    oh = jnp.where((iota_g == gidb) & (lane < _N), 1.0, 0.0)
    s_ref[...] += jnp.dot(oh, h, preferred_element_type=jnp.float32)
    c_ref[...] += jnp.dot(oh, jnp.ones((_TN, _H), jnp.float32),
                          preferred_element_type=jnp.float32)


def _readout(xp, neip, gidp, woa, wob, bo):
    grid = _NPAD // _TN
    return pl.pallas_call(
        _readout_body,
        grid=(grid,),
        in_specs=[
            pl.BlockSpec((_TN, _IN_NODE), lambda i: (i, 0)),
            pl.BlockSpec((_TN, _H), lambda i: (i, 0)),
            pl.BlockSpec((None, 1, _TN), lambda i: (i, 0, 0)),
            pl.BlockSpec((_IN_NODE, _H), lambda i: (0, 0)),
            pl.BlockSpec((_H, _H), lambda i: (0, 0)),
            pl.BlockSpec((1, _H), lambda i: (0, 0)),
        ],
        out_specs=[
            pl.BlockSpec((_GPAD, _H), lambda i: (0, 0)),
            pl.BlockSpec((_GPAD, _H), lambda i: (0, 0)),
        ],
        out_shape=[
            jax.ShapeDtypeStruct((_GPAD, _H), jnp.float32),
            jax.ShapeDtypeStruct((_GPAD, _H), jnp.float32),
        ],
    )(xp, neip, gidp, woa, wob, bo)


def kernel(x, edge_x, edge_index, tree_mess, tree_mess_source_edges,
           tree_mess_target_edges, graph_ids, W_i, W_h, W_o, b_o):
    src = edge_index[0]
    dst = edge_index[1]
    tmt = tree_mess_target_edges

    iotax = jnp.arange(_NX, dtype=jnp.int32)
    win = jnp.full((_E,), -1, jnp.int32).at[tmt].max(iotax)
    keep_b = win[tmt] == iotax
    rows = tree_mess[tree_mess_source_edges] * keep_b.astype(jnp.float32)[:, None]
    node_t = jnp.zeros((_N, _H), jnp.float32).at[dst[tmt]].add(rows)
    sidx = jnp.full((_E,), -1, jnp.int32).at[tmt ^ 1].max(
        jnp.where(keep_b, iotax, -1))
    idx_mod = jnp.where(sidx >= 0, _N + sidx, src)
    ext_tbl = jnp.concatenate([node_t, node_t[src[tmt ^ 1]] - rows], axis=0)
    zeros_nh = jnp.zeros((_N, _H), jnp.float32)
    src3 = src.reshape(_NW, _NCHUNK, _CH)
    dst3 = dst.reshape(_NW, _NCHUNK_S, _CH_S)

    aggt = _gather_rows(ext_tbl, idx_mod.reshape(_NW, _NCHUNK, _CH))
    bth, p = _prologue(edge_x, aggt, W_i, W_h)
    bth2 = bth.reshape(_E // 2, 2 * _H)

    for _ in range(_DEPTH - 2):
        parts = _scatter_add_rows(p, dst3, zeros_nh)
        tbl = parts[0] + parts[1]
        agg = _gather_rows(tbl, src3)
        p = _iter_step(p.reshape(_E // 2, 2 * _H), bth2,
                       agg.reshape(_E // 2, 2 * _H), W_h).reshape(_E, _H)

    parts = _scatter_add_rows(p, dst3, zeros_nh)
    tbl = parts[0] + parts[1]
    agg = _gather_rows(tbl, src3)
    g6 = _final_step(p.reshape(_E // 2, 2 * _H), bth2,
                     agg.reshape(_E // 2, 2 * _H)).reshape(_E, _H)

    parts = _scatter_add_rows(g6, dst3, zeros_nh)
    nei = parts[0] + parts[1] + node_t

    xp = jnp.pad(x, ((0, _NPAD - _N), (0, 0)))
    neip = jnp.pad(nei, ((0, _NPAD - _N), (0, 0)))
    gidp = jnp.pad(graph_ids, (0, _NPAD - _N),
                   constant_values=_NG).reshape(_NPAD // _TN, 1, _TN)
    s, c = _readout(xp, neip, gidp, W_o[:_IN_NODE], W_o[_IN_NODE:],
                    b_o.reshape(1, _H))
    return s[:_NG] / jnp.maximum(c[:_NG, :1], 1.0)
